# baseline (speedup 1.0000x reference)
from functools import partial

import jax
import jax.numpy as jnp
from jax.experimental import pallas as pl
from jax.experimental.pallas import tpu as pltpu

TOPK = 5
NC = 16
NEG = float("-inf")
POS = float("inf")


def _block_topk_body(q_ref, k_ref, v_out, i_out, *, bk, k_total, q):
    kb = pl.program_id(0)
    sims = jnp.dot(q_ref[...], k_ref[...].T,
                   preferred_element_type=jnp.float32)
    lidx = jax.lax.broadcasted_iota(jnp.int32, (q, bk), 1).astype(jnp.float32)
    nvalid = (k_total - kb * bk).astype(jnp.float32)
    sims = jnp.where(lidx < nvalid, sims, NEG)
    base = (kb * bk).astype(jnp.float32)
    vcols, icols = [], []
    for _ in range(TOPK):
        m = jnp.max(sims, axis=1, keepdims=True)
        z = jnp.where(sims == m, lidx, POS)
        sims = jnp.where(sims == m, NEG, sims)
        c = jnp.min(z, axis=1, keepdims=True)
        c2 = jnp.min(jnp.where(z == c, POS, z), axis=1, keepdims=True)
        vcols.append(m)
        icols.append(c + base)
        vcols.append(jnp.where(c2 == POS, NEG, m))
        icols.append(c2 + base)
    for _ in range(NC - 2 * TOPK):
        vcols.append(jnp.full((q, 1), NEG, jnp.float32))
        icols.append(jnp.full((q, 1), POS, jnp.float32))
    v_out[0, :, :] = jnp.concatenate(vcols, axis=1)
    i_out[0, :, :] = jnp.concatenate(icols, axis=1)


def _merge_topk_body(cv_ref, ci_ref, tv_ref, ti_ref, *, q):
    v = cv_ref[...]
    idx = ci_ref[...]
    vcols, icols = [], []
    for _ in range(TOPK):
        m = jnp.max(v, axis=1, keepdims=True)
        c = jnp.min(jnp.where(v == m, idx, POS), axis=1, keepdims=True)
        v = jnp.where(idx == c, NEG, v)
        vcols.append(m)
        icols.append(c)
    for _ in range(NC - TOPK):
        vcols.append(jnp.full((q, 1), NEG, jnp.float32))
        icols.append(jnp.full((q, 1), POS, jnp.float32))
    tv_ref[...] = jnp.concatenate(vcols, axis=1)
    ti_ref[...] = jnp.concatenate(icols, axis=1).astype(jnp.int32)


def kernel(queries, keys):
    q, d = queries.shape
    k_total = keys.shape[0]
    bk = 4096
    nkb = -(-k_total // bk)

    cand_v, cand_i = pl.pallas_call(
        partial(_block_topk_body, bk=bk, k_total=k_total, q=q),
        grid=(nkb,),
        in_specs=[
            pl.BlockSpec((q, d), lambda i: (0, 0)),
            pl.BlockSpec((bk, d), lambda i: (i, 0)),
        ],
        out_specs=[
            pl.BlockSpec((1, q, NC), lambda i: (i, 0, 0)),
            pl.BlockSpec((1, q, NC), lambda i: (i, 0, 0)),
        ],
        out_shape=[
            jax.ShapeDtypeStruct((nkb, q, NC), jnp.float32),
            jax.ShapeDtypeStruct((nkb, q, NC), jnp.float32),
        ],
        compiler_params=pltpu.CompilerParams(
            dimension_semantics=("arbitrary",)),
    )(queries, keys)

    width = nkb * NC
    cv = cand_v.transpose(1, 0, 2).reshape(q, width)
    ci = cand_i.transpose(1, 0, 2).reshape(q, width)

    tv, ti = pl.pallas_call(
        partial(_merge_topk_body, q=q),
        in_specs=[
            pl.BlockSpec((q, width), lambda: (0, 0)),
            pl.BlockSpec((q, width), lambda: (0, 0)),
        ],
        out_specs=[
            pl.BlockSpec((q, NC), lambda: (0, 0)),
            pl.BlockSpec((q, NC), lambda: (0, 0)),
        ],
        out_shape=[
            jax.ShapeDtypeStruct((q, NC), jnp.float32),
            jax.ShapeDtypeStruct((q, NC), jnp.int32),
        ],
    )(cv, ci)
    return tv[:, :TOPK], ti[:, :TOPK]


# speedup vs baseline: 1.3829x; 1.3829x over previous
from functools import partial

import jax
import jax.numpy as jnp
from jax.experimental import pallas as pl
from jax.experimental.pallas import tpu as pltpu

TOPK = 5
NC = 8
NEG = float("-inf")
POS = float("inf")


def _block_topk_body(q_ref, k_ref, v_out, i_out, *, bk, k_total, q):
    kb = pl.program_id(0)
    sims = jnp.dot(q_ref[...], k_ref[...].T,
                   preferred_element_type=jnp.float32)
    lidx = jax.lax.broadcasted_iota(jnp.int32, (q, bk), 1).astype(jnp.float32)
    nvalid = (k_total - kb * bk).astype(jnp.float32)
    sims = jnp.where(lidx < nvalid, sims, NEG)
    base = (kb * bk).astype(jnp.float32)
    vcols, icols = [], []
    for _ in range(TOPK):
        m = jnp.max(sims, axis=1, keepdims=True)
        c = jnp.min(jnp.where(sims == m, lidx, POS), axis=1, keepdims=True)
        sims = jnp.where(lidx == c, NEG, sims)
        vcols.append(m)
        icols.append(c + base)
    for _ in range(NC - TOPK):
        vcols.append(jnp.full((q, 1), NEG, jnp.float32))
        icols.append(jnp.full((q, 1), POS, jnp.float32))
    v_out[0, :, :] = jnp.concatenate(vcols, axis=1)
    i_out[0, :, :] = jnp.concatenate(icols, axis=1)


def _merge_topk_body(cv_ref, ci_ref, tv_ref, ti_ref, *, q):
    v = cv_ref[...]
    idx = ci_ref[...]
    vcols, icols = [], []
    for _ in range(TOPK):
        m = jnp.max(v, axis=1, keepdims=True)
        c = jnp.min(jnp.where(v == m, idx, POS), axis=1, keepdims=True)
        v = jnp.where(idx == c, NEG, v)
        vcols.append(m)
        icols.append(c)
    for _ in range(NC - TOPK):
        vcols.append(jnp.full((q, 1), NEG, jnp.float32))
        icols.append(jnp.full((q, 1), POS, jnp.float32))
    tv_ref[...] = jnp.concatenate(vcols, axis=1)
    ti_ref[...] = jnp.concatenate(icols, axis=1).astype(jnp.int32)


def kernel(queries, keys):
    q, d = queries.shape
    k_total = keys.shape[0]
    bk = 4096
    nkb = -(-k_total // bk)

    cand_v, cand_i = pl.pallas_call(
        partial(_block_topk_body, bk=bk, k_total=k_total, q=q),
        grid=(nkb,),
        in_specs=[
            pl.BlockSpec((q, d), lambda i: (0, 0)),
            pl.BlockSpec((bk, d), lambda i: (i, 0)),
        ],
        out_specs=[
            pl.BlockSpec((1, q, NC), lambda i: (i, 0, 0)),
            pl.BlockSpec((1, q, NC), lambda i: (i, 0, 0)),
        ],
        out_shape=[
            jax.ShapeDtypeStruct((nkb, q, NC), jnp.float32),
            jax.ShapeDtypeStruct((nkb, q, NC), jnp.float32),
        ],
        compiler_params=pltpu.CompilerParams(
            dimension_semantics=("arbitrary",)),
    )(queries, keys)

    width = nkb * NC
    cv = cand_v.transpose(1, 0, 2).reshape(q, width)
    ci = cand_i.transpose(1, 0, 2).reshape(q, width)

    tv, ti = pl.pallas_call(
        partial(_merge_topk_body, q=q),
        in_specs=[
            pl.BlockSpec((q, width), lambda: (0, 0)),
            pl.BlockSpec((q, width), lambda: (0, 0)),
        ],
        out_specs=[
            pl.BlockSpec((q, NC), lambda: (0, 0)),
            pl.BlockSpec((q, NC), lambda: (0, 0)),
        ],
        out_shape=[
            jax.ShapeDtypeStruct((q, NC), jnp.float32),
            jax.ShapeDtypeStruct((q, NC), jnp.int32),
        ],
    )(cv, ci)
    return tv[:, :TOPK], ti[:, :TOPK]
